# baseline (speedup 1.0000x reference)
import functools
import jax
import jax.numpy as jnp
from jax import lax
from jax.experimental import pallas as pl
from jax.experimental.pallas import tpu as pltpu
from jax.experimental.pallas import tpu_sc as plsc

QUARTERS = 4


@functools.cache
def _build(B, V, D):
    info = plsc.get_sparse_core_info()
    nw = info.num_cores * info.num_subcores
    assert D == nw
    q = B // QUARTERS
    mesh = plsc.VectorSubcoreMesh(core_axis_name="c", subcore_axis_name="s")

    @functools.partial(
        pl.kernel,
        mesh=mesh,
        out_type=jax.ShapeDtypeStruct((D, B), jnp.float32),
        compiler_params=pltpu.CompilerParams(
            use_tc_tiling_on_sc=True, needs_layout_passes=False
        ),
        scratch_types=[
            pltpu.VMEM((V,), jnp.float32),
            pltpu.VMEM((B,), jnp.int32),
            pltpu.VMEM((2, q), jnp.float32),
            pltpu.SemaphoreType.DMA,
            [pltpu.SemaphoreType.DMA] * QUARTERS,
            pltpu.SemaphoreType.DMA,
        ],
    )
    def k(idx_hbm, tablet_hbm, outt_hbm, row_v, idx_v, out_v, sem_row,
          sem_idx, sem_out):
        d = lax.axis_index("s") * info.num_cores + lax.axis_index("c")
        idxcps = [
            pltpu.make_async_copy(
                idx_hbm.at[pl.ds(c * q, q)],
                idx_v.at[pl.ds(c * q, q)],
                sem_idx[c],
            )
            for c in range(QUARTERS)
        ]
        idxcps[0].start()
        rowcp = pltpu.make_async_copy(tablet_hbm.at[d], row_v, sem_row)
        rowcp.start()
        for c in range(1, QUARTERS):
            idxcps[c].start()
        idxcps[0].wait()
        rowcp.wait()

        outcps = [None, None]
        for c in range(QUARTERS):
            buf = c % 2
            if c > 0:
                idxcps[c].wait()
            if outcps[buf] is not None:
                outcps[buf].wait()

            def body(sg, carry, c=c, buf=buf):
                prev = carry
                st = jnp.maximum(sg - 1, 0) * 128
                for l in range(8):
                    out_v[buf, pl.ds(st + l * 16, 16)] = prev[l]
                vecs = [
                    idx_v[pl.ds(c * q + sg * 128 + l * 16, 16)]
                    for l in range(8)
                ]
                return tuple(plsc.load_gather(row_v, [v]) for v in vecs)

            zero = jnp.zeros((16,), jnp.float32)
            last = lax.fori_loop(
                0, q // 128, body, (zero,) * 8, unroll=8
            )
            for l in range(8):
                out_v[buf, pl.ds(q - 128 + l * 16, 16)] = last[l]
            cp = pltpu.make_async_copy(
                out_v.at[buf], outt_hbm.at[d, pl.ds(c * q, q)], sem_out
            )
            cp.start()
            outcps[buf] = cp
        for cp in outcps:
            cp.wait()

    return k


def kernel(nombre, table):
    B = nombre.shape[0]
    V, D = table.shape
    idx = nombre.astype(jnp.int32)
    outt = _build(B, V, D)(idx, table.T)
    return outt.T


# speedup vs baseline: 1.0093x; 1.0093x over previous
import functools
import jax
import jax.numpy as jnp
from jax import lax
from jax.experimental import pallas as pl
from jax.experimental.pallas import tpu as pltpu
from jax.experimental.pallas import tpu_sc as plsc

QUARTERS = 4


@functools.cache
def _build(B, V, D):
    info = plsc.get_sparse_core_info()
    nw = info.num_cores * info.num_subcores
    assert D == nw
    q = B // QUARTERS
    mesh = plsc.VectorSubcoreMesh(core_axis_name="c", subcore_axis_name="s")

    @functools.partial(
        pl.kernel,
        mesh=mesh,
        out_type=jax.ShapeDtypeStruct((D, B), jnp.float32),
        compiler_params=pltpu.CompilerParams(
            use_tc_tiling_on_sc=True, needs_layout_passes=False
        ),
        scratch_types=[
            pltpu.VMEM((V,), jnp.float32),
            pltpu.VMEM((B,), jnp.int32),
            pltpu.VMEM((2, q), jnp.float32),
            pltpu.SemaphoreType.DMA,
            [pltpu.SemaphoreType.DMA] * QUARTERS,
            pltpu.SemaphoreType.DMA,
        ],
    )
    def k(idx_hbm, tablet_hbm, outt_hbm, row_v, idx_v, out_v, sem_row,
          sem_idx, sem_out):
        d = lax.axis_index("s") * info.num_cores + lax.axis_index("c")
        idxcps = [
            pltpu.make_async_copy(
                idx_hbm.at[pl.ds(c * q, q)],
                idx_v.at[pl.ds(c * q, q)],
                sem_idx[c],
            )
            for c in range(QUARTERS)
        ]
        idxcps[0].start()
        rowcp = pltpu.make_async_copy(tablet_hbm.at[d], row_v, sem_row)
        rowcp.start()
        for c in range(1, QUARTERS):
            idxcps[c].start()
        idxcps[0].wait()
        rowcp.wait()

        outcps = [None, None]
        for c in range(QUARTERS):
            buf = c % 2
            if c > 0:
                idxcps[c].wait()
            if outcps[buf] is not None:
                outcps[buf].wait()

            def body(sg, carry, c=c, buf=buf):
                vecs = [
                    idx_v[pl.ds(c * q + sg * 128 + l * 16, 16)]
                    for l in range(8)
                ]
                vals = [plsc.load_gather(row_v, [v]) for v in vecs]
                for l in range(8):
                    out_v[buf, pl.ds(sg * 128 + l * 16, 16)] = vals[l]
                return carry

            lax.fori_loop(0, q // 128, body, 0, unroll=8)
            cp = pltpu.make_async_copy(
                out_v.at[buf], outt_hbm.at[d, pl.ds(c * q, q)], sem_out
            )
            cp.start()
            outcps[buf] = cp
        for cp in outcps:
            cp.wait()

    return k


def kernel(nombre, table):
    B = nombre.shape[0]
    V, D = table.shape
    idx = nombre.astype(jnp.int32)
    outt = _build(B, V, D)(idx, table.T)
    return outt.T
